# Initial kernel scaffold; baseline (speedup 1.0000x reference)
import jax
import jax.numpy as jnp
from jax.experimental import pallas as pl


def kernel(node_feats, edge_feats, edge_index, Wq, Wk, Wv, We):
    raise NotImplementedError("write your pallas kernel here")


# speedup vs baseline: 11.7259x; 11.7259x over previous
import functools

import jax
import jax.numpy as jnp
from jax import lax
from jax.experimental import pallas as pl
from jax.experimental.pallas import tpu as pltpu
from jax.experimental.pallas import tpu_sc as plsc

_N = 10000
_E = 320000
_F = 128
_H = 8
_D = 16
_HD = _H * _D

_NC = 2
_NS = 16
_NW = _NC * _NS
_EW = _E // _NW
_C = 64
_NCHUNK = 160
_TAIL = 156
_EWP = _NCHUNK * _C
_NPW = 10240
_NPZ = _NPW // 8
_NACC = _NPW + _NPZ
_RPT = _NACC // _NS


def _nproj_body(x_ref, wkv_ref, wq_ref, kv_ref, q_ref):
    x = x_ref[...]
    kv_ref[...] = jnp.dot(x, wkv_ref[...], preferred_element_type=jnp.float32)
    q_ref[...] = jnp.dot(x, wq_ref[...], preferred_element_type=jnp.float32)


def _node_proj(node_feats, wkv, wq):
    blk = 2000
    grid = _N // blk
    return pl.pallas_call(
        _nproj_body,
        grid=(grid,),
        in_specs=[
            pl.BlockSpec((blk, _F), lambda i: (i, 0)),
            pl.BlockSpec((_F, 2 * _HD), lambda i: (0, 0)),
            pl.BlockSpec((_F, _HD), lambda i: (0, 0)),
        ],
        out_specs=[
            pl.BlockSpec((blk, 2 * _HD), lambda i: (i, 0)),
            pl.BlockSpec((blk, _HD), lambda i: (i, 0)),
        ],
        out_shape=[
            jax.ShapeDtypeStruct((_N, 2 * _HD), jnp.float32),
            jax.ShapeDtypeStruct((_N, _HD), jnp.float32),
        ],
    )(node_feats, wkv, wq)


def _eproj_body(x_ref, w_ref, o_ref):
    o_ref[...] = jnp.dot(x_ref[...], w_ref[...], preferred_element_type=jnp.float32)


def _edge_proj(edge_feats, we):
    blk = 2000
    grid = _E // blk
    return pl.pallas_call(
        _eproj_body,
        grid=(grid,),
        in_specs=[
            pl.BlockSpec((blk, _F), lambda i: (i, 0)),
            pl.BlockSpec((_F, _HD), lambda i: (0, 0)),
        ],
        out_specs=pl.BlockSpec((blk, _HD), lambda i: (i, 0)),
        out_shape=jax.ShapeDtypeStruct((_E, _HD), jnp.float32),
    )(edge_feats, we)


def _edge_kernel(kv_hbm, q_hbm, pe_hbm, src_hbm, dst_hbm,
                 eout_hbm, acc_hbm,
                 idx_s, idx_d, idx_z, kvb, qb, eo, s0, s1, acc_sh):
    cid = lax.axis_index("c")
    sid = lax.axis_index("s")
    wid = sid * _NC + cid
    row0 = pl.multiple_of(sid * _RPT, 8)

    lane = lax.iota(jnp.int32, 16)
    zeros16 = jnp.zeros((16,), jnp.float32)
    perms = [(lane ^ k).reshape(16, 1) for k in (8, 4, 2, 1)]
    dnums = lax.GatherDimensionNumbers(
        offset_dims=(), collapsed_slice_dims=(0,), start_index_map=(0,))

    def _allsum(v):
        for p in perms:
            v = v + lax.gather(v, p, dnums, (1,),
                               mode=lax.GatherScatterMode.PROMISE_IN_BOUNDS)
        return v

    def _zrow(r, c2):
        for cc in range(8):
            qb[r, pl.ds(cc * 16, 16)] = zeros16
        return c2
    lax.fori_loop(0, _C, _zrow, 0)

    def _init(k, c2):
        r0 = pl.multiple_of(row0 + k * 48, 8)
        pltpu.sync_copy(qb.at[pl.ds(0, 48)], acc_sh.at[pl.ds(r0, 48)])
        return c2
    lax.fori_loop(0, _RPT // 48, _init, 0)
    plsc.subcore_barrier()

    def chunk(i, carry):
        pbase = pl.multiple_of(wid * _EWP + i * _C, 8)
        rbase = pl.multiple_of(wid * _EW + i * _C, 8)
        tbase = pl.multiple_of(wid * _EW + _TAIL * _C, 8)
        pltpu.sync_copy(src_hbm.at[pl.ds(pbase, _C)], idx_s)
        pltpu.sync_copy(dst_hbm.at[pl.ds(pbase, _C)], idx_d)
        for t in range(_C // 16):
            sl = pl.ds(t * 16, 16)
            idx_z[sl] = (idx_d[sl] >> 3) + _NPW
        c1 = pltpu.async_copy(kv_hbm.at[idx_s], kvb, s0)
        c2 = pltpu.async_copy(q_hbm.at[idx_d], qb, s1)
        c1.wait()
        c2.wait()

        @pl.when(i < _TAIL)
        def _():
            pltpu.sync_copy(pe_hbm.at[pl.ds(rbase, _C)], eo)

        @pl.when(i == _TAIL)
        def _():
            pltpu.sync_copy(pe_hbm.at[pl.ds(tbase, 16)], eo.at[pl.ds(0, 16)])

        def edge(e, c):
            live = jnp.where(i * _C + e < _EW, 1.0, 0.0)
            sv = zeros16
            for h in range(_H):
                sl = pl.ds(h * _D, _D)
                sc = (jnp.clip(kvb[e, sl] * qb[e, sl] * 0.25, -5.0, 5.0)
                      * eo[e, sl])
                eo[e, sl] = sc
                w = jnp.exp(jnp.clip(_allsum(sc), -5.0, 5.0)) * live
                qb[e, sl] = kvb[e, pl.ds(_HD + h * _D, _D)] * w
                sv = jnp.where(lane == h, w, sv)
            kvb[e, pl.ds(0, 16)] = sv
            return c

        lax.fori_loop(0, _C, edge, 0)

        @pl.when(i < _TAIL)
        def _():
            pltpu.sync_copy(eo, eout_hbm.at[pl.ds(rbase, _C)])

        @pl.when(i == _TAIL)
        def _():
            pltpu.sync_copy(eo.at[pl.ds(0, 16)], eout_hbm.at[pl.ds(tbase, 16)])

        pltpu.sync_copy(qb, acc_sh.at[idx_d], add=True)

        def zrow(g, c):
            dvec = idx_d[pl.ds(g * 16, 16)] & 7
            for j in range(16):
                e = g * 16 + j
                m = dvec[j]
                sv = kvb[e, pl.ds(0, 16)]
                for b in range(8):
                    eo[e, pl.ds(b * 16, 16)] = jnp.where(m == b, sv, zeros16)
            return c
        lax.fori_loop(0, _C // 16, zrow, 0)
        pltpu.sync_copy(eo, acc_sh.at[idx_z], add=True)
        return carry

    lax.fori_loop(0, _NCHUNK, chunk, 0)

    plsc.subcore_barrier()
    out0 = pl.multiple_of(cid * _NACC + row0, 8)

    def _out(k, c2):
        r0 = pl.multiple_of(row0 + k * 48, 8)
        o0 = pl.multiple_of(out0 + k * 48, 8)
        pltpu.sync_copy(acc_sh.at[pl.ds(r0, 48)], qb.at[pl.ds(0, 48)])
        pltpu.sync_copy(qb.at[pl.ds(0, 48)], acc_hbm.at[pl.ds(o0, 48)])
        return c2
    lax.fori_loop(0, _RPT // 48, _out, 0)


_edge_attention = functools.partial(
    pl.kernel,
    out_type=(
        jax.ShapeDtypeStruct((_E, _HD), jnp.float32),
        jax.ShapeDtypeStruct((_NC * _NACC, _HD), jnp.float32),
    ),
    mesh=plsc.VectorSubcoreMesh(core_axis_name="c", subcore_axis_name="s"),
    scratch_types=[
        pltpu.VMEM((_C,), jnp.int32),
        pltpu.VMEM((_C,), jnp.int32),
        pltpu.VMEM((_C,), jnp.int32),
        pltpu.VMEM((_C, 2 * _HD), jnp.float32),
        pltpu.VMEM((_C, _HD), jnp.float32),
        pltpu.VMEM((_C, _HD), jnp.float32),
        pltpu.SemaphoreType.DMA,
        pltpu.SemaphoreType.DMA,
        pltpu.VMEM_SHARED((_NACC, _HD), jnp.float32),
    ],
)(_edge_kernel)


def _combine_body(wv_ref, z_ref, ex_ref, o_ref):
    wv = wv_ref[0] + wv_ref[1]
    zz = z_ref[0] + z_ref[1]
    den = jnp.dot(zz, ex_ref[...], preferred_element_type=jnp.float32) + 1e-6
    o_ref[...] = wv / den


def _combine(wv_part, z_part, expand):
    blk = 2000
    grid = _N // blk
    return pl.pallas_call(
        _combine_body,
        grid=(grid,),
        in_specs=[
            pl.BlockSpec((2, blk, _HD), lambda i: (0, i, 0)),
            pl.BlockSpec((2, blk, _H), lambda i: (0, i, 0)),
            pl.BlockSpec((_H, _HD), lambda i: (0, 0)),
        ],
        out_specs=pl.BlockSpec((blk, _HD), lambda i: (i, 0)),
        out_shape=jax.ShapeDtypeStruct((_N, _HD), jnp.float32),
    )(wv_part, z_part, expand)


def kernel(node_feats, edge_feats, edge_index, Wq, Wk, Wv, We):
    src_p = jnp.pad(edge_index[0].reshape(_NW, _EW),
                    ((0, 0), (0, _EWP - _EW))).reshape(-1)
    dst_p = jnp.pad(edge_index[1].reshape(_NW, _EW),
                    ((0, 0), (0, _EWP - _EW))).reshape(-1)
    wkv = jnp.concatenate([Wk, Wv], axis=1)

    kv, q = _node_proj(node_feats, wkv, Wq)
    pe = _edge_proj(edge_feats, We)

    e_out, acc = _edge_attention(kv, q, pe, src_p, dst_p)

    acc = acc.reshape(2, _NACC, _HD)
    wv_part = acc[:, :_N]
    z_part = (acc[:, _NPW:_NPW + _N // 8]
              .reshape(2, _N // 8, 8, 16)[:, :, :, :_H]
              .reshape(2, _N, _H))
    expand = (jnp.arange(_HD)[None, :] // _D == jnp.arange(_H)[:, None]
              ).astype(jnp.float32)
    h_out = _combine(wv_part, z_part, expand)

    return h_out.reshape(_N, _H, _D), e_out.reshape(_E, _H, _D)


# speedup vs baseline: 12.2785x; 1.0471x over previous
import functools

import jax
import jax.numpy as jnp
from jax import lax
from jax.experimental import pallas as pl
from jax.experimental.pallas import tpu as pltpu
from jax.experimental.pallas import tpu_sc as plsc

_N = 10000
_E = 320000
_F = 128
_H = 8
_D = 16
_HD = _H * _D

_NC = 2
_NS = 16
_NW = _NC * _NS
_EW = _E // _NW
_C = 24
_NCHUNK = 424
_TAIL = 416
_EWP = _NCHUNK * _C
_EPAD = 322000
_NPW = 10240
_NPZ = _NPW // 8
_NACC = _NPW + _NPZ
_RPT = _NACC // _NS


def _nproj_body(x_ref, wkv_ref, wq_ref, kv_ref, q_ref):
    x = x_ref[...]
    kv_ref[...] = jnp.dot(x, wkv_ref[...], preferred_element_type=jnp.float32)
    q_ref[...] = jnp.dot(x, wq_ref[...], preferred_element_type=jnp.float32)


def _node_proj(node_feats, wkv, wq):
    blk = 2000
    grid = _N // blk
    return pl.pallas_call(
        _nproj_body,
        grid=(grid,),
        in_specs=[
            pl.BlockSpec((blk, _F), lambda i: (i, 0)),
            pl.BlockSpec((_F, 2 * _HD), lambda i: (0, 0)),
            pl.BlockSpec((_F, _HD), lambda i: (0, 0)),
        ],
        out_specs=[
            pl.BlockSpec((blk, 2 * _HD), lambda i: (i, 0)),
            pl.BlockSpec((blk, _HD), lambda i: (i, 0)),
        ],
        out_shape=[
            jax.ShapeDtypeStruct((_N, 2 * _HD), jnp.float32),
            jax.ShapeDtypeStruct((_N, _HD), jnp.float32),
        ],
    )(node_feats, wkv, wq)


def _eproj_body(x_ref, w_ref, o_ref):
    o_ref[...] = jnp.dot(x_ref[...], w_ref[...], preferred_element_type=jnp.float32)


def _edge_proj(edge_feats, we):
    blk = 2000
    grid = _EPAD // blk
    return pl.pallas_call(
        _eproj_body,
        grid=(grid,),
        in_specs=[
            pl.BlockSpec((blk, _F), lambda i: (jnp.minimum(i, _E // blk - 1), 0)),
            pl.BlockSpec((_F, _HD), lambda i: (0, 0)),
        ],
        out_specs=pl.BlockSpec((blk, _HD), lambda i: (i, 0)),
        out_shape=jax.ShapeDtypeStruct((_EPAD, _HD), jnp.float32),
    )(edge_feats, we)


def _edge_kernel(kv_hbm, q_hbm, pe_hbm, src_hbm, dst_hbm,
                 eout_hbm, acc_hbm,
                 irs0, ird0, irs1, ird1,
                 ids0, idz0, ids1, idz1,
                 kvb0, qb0, eo0, zb0,
                 kvb1, qb1, eo1, zb1,
                 sg0, sg1, so0, so1, si0, si1,
                 acc_sh):
    cid = lax.axis_index("c")
    sid = lax.axis_index("s")
    wid = sid * _NC + cid
    row0 = pl.multiple_of(sid * _RPT, 8)

    irs = [irs0, irs1]
    ird = [ird0, ird1]
    ids = [ids0, ids1]
    idz = [idz0, idz1]
    kvb = [kvb0, kvb1]
    qb = [qb0, qb1]
    eo = [eo0, eo1]
    zb = [zb0, zb1]
    sg = [sg0, sg1]
    so = [so0, so1]
    si = [si0, si1]

    lane = lax.iota(jnp.int32, 16)
    zeros16 = jnp.zeros((16,), jnp.float32)
    perms = [(lane ^ k).reshape(16, 1) for k in (8, 4, 2, 1)]
    dnums = lax.GatherDimensionNumbers(
        offset_dims=(), collapsed_slice_dims=(0,), start_index_map=(0,))

    def _allsum(v):
        for p in perms:
            v = v + lax.gather(v, p, dnums, (1,),
                               mode=lax.GatherScatterMode.PROMISE_IN_BOUNDS)
        return v

    def _zrow(r, c2):
        for cc in range(8):
            qb0[r, pl.ds(cc * 16, 16)] = zeros16
        return c2
    lax.fori_loop(0, _C, _zrow, 0)

    def _init(k, c2):
        r0 = pl.multiple_of(row0 + k * _C, 8)
        pltpu.sync_copy(qb0, acc_sh.at[pl.ds(r0, _C)])
        return c2
    lax.fori_loop(0, _RPT // _C, _init, 0)
    plsc.subcore_barrier()

    def _issue_idx(p, slot):
        pb = pl.multiple_of(wid * _EWP + p * _C, 8)
        pltpu.async_copy(src_hbm.at[pl.ds(pb, _C)], irs[slot], si[slot])
        pltpu.async_copy(dst_hbm.at[pl.ds(pb, _C)], ird[slot], si[slot])

    def _wait_idx(slot):
        pb0 = pl.multiple_of(wid * _EWP, 8)
        pltpu.make_async_copy(src_hbm.at[pl.ds(pb0, _C)], irs[slot],
                              si[slot]).wait()
        pltpu.make_async_copy(dst_hbm.at[pl.ds(pb0, _C)], ird[slot],
                              si[slot]).wait()

    def _issue_gathers(p, s):
        rb = pl.multiple_of(wid * _EW + jnp.minimum(p, _TAIL) * _C, 8)
        pltpu.async_copy(kv_hbm.at[irs[s]], kvb[s], sg[s])
        pltpu.async_copy(q_hbm.at[ird[s]], qb[s], sg[s])
        pltpu.async_copy(pe_hbm.at[pl.ds(rb, _C)], eo[s], sg[s])

    def _wait_gathers(s):
        rb0 = pl.multiple_of(wid * _EW, 8)
        pltpu.make_async_copy(kv_hbm.at[irs[s]], kvb[s], sg[s]).wait()
        pltpu.make_async_copy(q_hbm.at[ird[s]], qb[s], sg[s]).wait()
        pltpu.make_async_copy(pe_hbm.at[pl.ds(rb0, _C)], eo[s], sg[s]).wait()

    def _issue_outputs(p, s):
        rb = pl.multiple_of(wid * _EW + jnp.minimum(p, _TAIL) * _C, 8)

        @pl.when(p < _TAIL)
        def _():
            pltpu.async_copy(eo[s], eout_hbm.at[pl.ds(rb, _C)], so[s])

        @pl.when(p == _TAIL)
        def _():
            pltpu.async_copy(eo[s].at[pl.ds(0, 16)],
                             eout_hbm.at[pl.ds(rb, 16)], so[s])

        pltpu.sync_copy(qb[s], acc_sh.at[ids[s]], add=True)
        pltpu.sync_copy(zb[s], acc_sh.at[idz[s]], add=True)

    def _wait_outputs(p, s):
        rb0 = pl.multiple_of(wid * _EW, 8)

        @pl.when(p < _TAIL)
        def _():
            pltpu.make_async_copy(eo[s], eout_hbm.at[pl.ds(rb0, _C)],
                                  so[s]).wait()

        @pl.when(p == _TAIL)
        def _():
            pltpu.make_async_copy(eo[s].at[pl.ds(0, 16)],
                                  eout_hbm.at[pl.ds(rb0, 16)], so[s]).wait()

        pass

    def _compute(p, s):
        for b in (0, 8):
            dv = ird[s][pl.ds(b, 16)]
            ids[s][pl.ds(b, 16)] = dv
            idz[s][pl.ds(b, 16)] = (dv >> 3) + _NPW

        def edge(e, c):
            live = jnp.where(p * _C + e < _EW, 1.0, 0.0)
            sv = zeros16
            for h in range(_H):
                sl = pl.ds(h * _D, _D)
                sc = (jnp.clip(kvb[s][e, sl] * qb[s][e, sl] * 0.25, -5.0, 5.0)
                      * eo[s][e, sl])
                eo[s][e, sl] = sc
                w = jnp.exp(jnp.clip(_allsum(sc), -5.0, 5.0)) * live
                qb[s][e, sl] = kvb[s][e, pl.ds(_HD + h * _D, _D)] * w
                sv = jnp.where(lane == h, w, sv)
            kvb[s][e, pl.ds(0, 16)] = sv
            return c
        lax.fori_loop(0, _C, edge, 0)

        for b in (0, 8):
            dvec = ids[s][pl.ds(b, 16)] & 7
            for j in range(16):
                e = b + j
                m = dvec[j]
                sv = kvb[s][e, pl.ds(0, 16)]
                for blk in range(8):
                    zb[s][e, pl.ds(blk * 16, 16)] = jnp.where(
                        m == blk, sv, zeros16)

    def _chunk_step(p, s):
        _wait_gathers(s)
        _compute(p, s)
        _issue_outputs(p, s)

        @pl.when(p > 0)
        def _():
            _wait_outputs(p - 1, 1 - s)

        @pl.when(p < _NCHUNK - 1)
        def _():
            _wait_idx(1 - s)
            _issue_gathers(p + 1, 1 - s)

        @pl.when(p < _NCHUNK - 2)
        def _():
            _issue_idx(p + 2, s)

    pb0 = pl.multiple_of(wid * _EWP, 8)
    pltpu.sync_copy(src_hbm.at[pl.ds(pb0, _C)], irs0)
    pltpu.sync_copy(dst_hbm.at[pl.ds(pb0, _C)], ird0)
    _issue_gathers(0, 0)
    _issue_idx(1, 1)

    def body(t, carry):
        _chunk_step(2 * t, 0)
        _chunk_step(2 * t + 1, 1)
        return carry
    lax.fori_loop(0, _NCHUNK // 2, body, 0)

    _wait_outputs(_NCHUNK - 1, 1)

    plsc.subcore_barrier()
    out0 = pl.multiple_of(cid * _NACC + row0, 8)

    def _out(k, c2):
        r0 = pl.multiple_of(row0 + k * _C, 8)
        o0 = pl.multiple_of(out0 + k * _C, 8)
        pltpu.sync_copy(acc_sh.at[pl.ds(r0, _C)], qb0)
        pltpu.sync_copy(qb0, acc_hbm.at[pl.ds(o0, _C)])
        return c2
    lax.fori_loop(0, _RPT // _C, _out, 0)


_edge_attention = functools.partial(
    pl.kernel,
    out_type=(
        jax.ShapeDtypeStruct((_E, _HD), jnp.float32),
        jax.ShapeDtypeStruct((_NC * _NACC, _HD), jnp.float32),
    ),
    mesh=plsc.VectorSubcoreMesh(core_axis_name="c", subcore_axis_name="s"),
    scratch_types=[
        pltpu.VMEM((_C,), jnp.int32),
        pltpu.VMEM((_C,), jnp.int32),
        pltpu.VMEM((_C,), jnp.int32),
        pltpu.VMEM((_C,), jnp.int32),
        pltpu.VMEM((_C,), jnp.int32),
        pltpu.VMEM((_C,), jnp.int32),
        pltpu.VMEM((_C,), jnp.int32),
        pltpu.VMEM((_C,), jnp.int32),
        pltpu.VMEM((_C, 2 * _HD), jnp.float32),
        pltpu.VMEM((_C, _HD), jnp.float32),
        pltpu.VMEM((_C, _HD), jnp.float32),
        pltpu.VMEM((_C, _HD), jnp.float32),
        pltpu.VMEM((_C, 2 * _HD), jnp.float32),
        pltpu.VMEM((_C, _HD), jnp.float32),
        pltpu.VMEM((_C, _HD), jnp.float32),
        pltpu.VMEM((_C, _HD), jnp.float32),
        pltpu.SemaphoreType.DMA,
        pltpu.SemaphoreType.DMA,
        pltpu.SemaphoreType.DMA,
        pltpu.SemaphoreType.DMA,
        pltpu.SemaphoreType.DMA,
        pltpu.SemaphoreType.DMA,
        pltpu.VMEM_SHARED((_NACC, _HD), jnp.float32),
    ],
)(_edge_kernel)


def _combine_body(wv_ref, z_ref, ex_ref, o_ref):
    wv = wv_ref[0] + wv_ref[1]
    zz = z_ref[0] + z_ref[1]
    den = jnp.dot(zz, ex_ref[...], preferred_element_type=jnp.float32) + 1e-6
    o_ref[...] = wv / den


def _combine(wv_part, z_part, expand):
    blk = 2000
    grid = _N // blk
    return pl.pallas_call(
        _combine_body,
        grid=(grid,),
        in_specs=[
            pl.BlockSpec((2, blk, _HD), lambda i: (0, i, 0)),
            pl.BlockSpec((2, blk, _H), lambda i: (0, i, 0)),
            pl.BlockSpec((_H, _HD), lambda i: (0, 0)),
        ],
        out_specs=pl.BlockSpec((blk, _HD), lambda i: (i, 0)),
        out_shape=jax.ShapeDtypeStruct((_N, _HD), jnp.float32),
    )(wv_part, z_part, expand)


def kernel(node_feats, edge_feats, edge_index, Wq, Wk, Wv, We):
    src_p = jnp.pad(edge_index[0].reshape(_NW, _EW),
                    ((0, 0), (0, _EWP - _EW))).reshape(-1)
    dst_p = jnp.pad(edge_index[1].reshape(_NW, _EW),
                    ((0, 0), (0, _EWP - _EW))).reshape(-1)
    wkv = jnp.concatenate([Wk, Wv], axis=1)

    kv, q = _node_proj(node_feats, wkv, Wq)
    pe = _edge_proj(edge_feats, We)

    e_out, acc = _edge_attention(kv, q, pe, src_p, dst_p)

    acc = acc.reshape(2, _NACC, _HD)
    wv_part = acc[:, :_N]
    z_part = (acc[:, _NPW:_NPW + _N // 8]
              .reshape(2, _N // 8, 8, 16)[:, :, :, :_H]
              .reshape(2, _N, _H))
    expand = (jnp.arange(_HD)[None, :] // _D == jnp.arange(_H)[:, None]
              ).astype(jnp.float32)
    h_out = _combine(wv_part, z_part, expand)

    return h_out.reshape(_N, _H, _D), e_out.reshape(_E, _H, _D)


# speedup vs baseline: 12.8135x; 1.0436x over previous
import functools

import jax
import jax.numpy as jnp
from jax import lax
from jax.experimental import pallas as pl
from jax.experimental.pallas import tpu as pltpu
from jax.experimental.pallas import tpu_sc as plsc

_N = 10000
_E = 320000
_F = 128
_H = 8
_D = 16
_HD = _H * _D

_NC = 2
_NS = 16
_NW = _NC * _NS
_EW = _E // _NW
_C = 24
_NCHUNK = 424
_TAIL = 416
_EWP = _NCHUNK * _C
_EPAD = 322000
_NPW = 10240
_NPZ = _NPW // 8
_NACC = _NPW + _NPZ
_RPT = _NACC // _NS


def _nproj_body(x_ref, wkv_ref, wq_ref, kv_ref, q_ref):
    x = x_ref[...]
    kv_ref[...] = jnp.dot(x, wkv_ref[...], preferred_element_type=jnp.float32)
    q_ref[...] = jnp.dot(x, wq_ref[...], preferred_element_type=jnp.float32)


def _node_proj(node_feats, wkv, wq):
    blk = 2000
    grid = _N // blk
    return pl.pallas_call(
        _nproj_body,
        grid=(grid,),
        in_specs=[
            pl.BlockSpec((blk, _F), lambda i: (i, 0)),
            pl.BlockSpec((_F, 2 * _HD), lambda i: (0, 0)),
            pl.BlockSpec((_F, _HD), lambda i: (0, 0)),
        ],
        out_specs=[
            pl.BlockSpec((blk, 2 * _HD), lambda i: (i, 0)),
            pl.BlockSpec((blk, _HD), lambda i: (i, 0)),
        ],
        out_shape=[
            jax.ShapeDtypeStruct((_N, 2 * _HD), jnp.float32),
            jax.ShapeDtypeStruct((_N, _HD), jnp.float32),
        ],
    )(node_feats, wkv, wq)


def _eproj_body(x_ref, w_ref, o_ref):
    o_ref[...] = jnp.dot(x_ref[...], w_ref[...], preferred_element_type=jnp.float32)


def _edge_proj(edge_feats, we):
    blk = 2000
    grid = _EPAD // blk
    return pl.pallas_call(
        _eproj_body,
        grid=(grid,),
        in_specs=[
            pl.BlockSpec((blk, _F), lambda i: (jnp.minimum(i, _E // blk - 1), 0)),
            pl.BlockSpec((_F, _HD), lambda i: (0, 0)),
        ],
        out_specs=pl.BlockSpec((blk, _HD), lambda i: (i, 0)),
        out_shape=jax.ShapeDtypeStruct((_EPAD, _HD), jnp.float32),
    )(edge_feats, we)


def _edge_kernel(kv_hbm, q_hbm, pe_hbm, src_hbm, dst_hbm,
                 eout_hbm, acc_hbm,
                 irs0, ird0, irs1, ird1,
                 ids0, idz0, ids1, idz1,
                 kvb0, qb0, eo0, zb0,
                 kvb1, qb1, eo1, zb1,
                 sg0, sg1, so0, so1, si0, si1, sa0, sa1,
                 acc_sh):
    cid = lax.axis_index("c")
    sid = lax.axis_index("s")
    wid = sid * _NC + cid
    row0 = pl.multiple_of(sid * _RPT, 8)

    irs = [irs0, irs1]
    ird = [ird0, ird1]
    ids = [ids0, ids1]
    idz = [idz0, idz1]
    kvb = [kvb0, kvb1]
    qb = [qb0, qb1]
    eo = [eo0, eo1]
    zb = [zb0, zb1]
    sg = [sg0, sg1]
    so = [so0, so1]
    si = [si0, si1]
    sa = [sa0, sa1]

    lane = lax.iota(jnp.int32, 16)
    zeros16 = jnp.zeros((16,), jnp.float32)
    perms = [(lane ^ k).reshape(16, 1) for k in (8, 4, 2, 1)]
    dnums = lax.GatherDimensionNumbers(
        offset_dims=(), collapsed_slice_dims=(0,), start_index_map=(0,))

    def _allsum(v):
        for p in perms:
            v = v + lax.gather(v, p, dnums, (1,),
                               mode=lax.GatherScatterMode.PROMISE_IN_BOUNDS)
        return v

    def _zrow(r, c2):
        for cc in range(8):
            qb0[r, pl.ds(cc * 16, 16)] = zeros16
        return c2
    lax.fori_loop(0, _C, _zrow, 0)

    def _init(k, c2):
        r0 = pl.multiple_of(row0 + k * _C, 8)
        pltpu.sync_copy(qb0, acc_sh.at[pl.ds(r0, _C)])
        return c2
    lax.fori_loop(0, _RPT // _C, _init, 0)
    plsc.subcore_barrier()

    def _issue_idx(p, slot):
        pb = pl.multiple_of(wid * _EWP + p * _C, 8)
        pltpu.async_copy(src_hbm.at[pl.ds(pb, _C)], irs[slot], si[slot])
        pltpu.async_copy(dst_hbm.at[pl.ds(pb, _C)], ird[slot], si[slot])

    def _wait_idx(slot):
        pb0 = pl.multiple_of(wid * _EWP, 8)
        pltpu.make_async_copy(src_hbm.at[pl.ds(pb0, _C)], irs[slot],
                              si[slot]).wait()
        pltpu.make_async_copy(dst_hbm.at[pl.ds(pb0, _C)], ird[slot],
                              si[slot]).wait()

    def _issue_gathers(p, s):
        rb = pl.multiple_of(wid * _EW + jnp.minimum(p, _TAIL) * _C, 8)
        pltpu.async_copy(kv_hbm.at[irs[s]], kvb[s], sg[s])
        pltpu.async_copy(q_hbm.at[ird[s]], qb[s], sg[s])
        pltpu.async_copy(pe_hbm.at[pl.ds(rb, _C)], eo[s], sg[s])

    def _wait_gathers(s):
        rb0 = pl.multiple_of(wid * _EW, 8)
        pltpu.make_async_copy(kv_hbm.at[irs[s]], kvb[s], sg[s]).wait()
        pltpu.make_async_copy(q_hbm.at[ird[s]], qb[s], sg[s]).wait()
        pltpu.make_async_copy(pe_hbm.at[pl.ds(rb0, _C)], eo[s], sg[s]).wait()

    def _issue_outputs(p, s):
        rb = pl.multiple_of(wid * _EW + jnp.minimum(p, _TAIL) * _C, 8)

        @pl.when(p < _TAIL)
        def _():
            pltpu.async_copy(eo[s], eout_hbm.at[pl.ds(rb, _C)], so[s])

        @pl.when(p == _TAIL)
        def _():
            pltpu.async_copy(eo[s].at[pl.ds(0, 16)],
                             eout_hbm.at[pl.ds(rb, 16)], so[s])

        pltpu.async_copy(qb[s], acc_sh.at[ids[s]], sa[s], add=True)
        pltpu.async_copy(zb[s], acc_sh.at[idz[s]], sa[s], add=True)

    def _wait_outputs(p, s):
        rb0 = pl.multiple_of(wid * _EW, 8)

        @pl.when(p < _TAIL)
        def _():
            pltpu.make_async_copy(eo[s], eout_hbm.at[pl.ds(rb0, _C)],
                                  so[s]).wait()

        @pl.when(p == _TAIL)
        def _():
            pltpu.make_async_copy(eo[s].at[pl.ds(0, 16)],
                                  eout_hbm.at[pl.ds(rb0, 16)], so[s]).wait()

        pltpu.make_async_copy(qb[s], acc_sh.at[ids[s]], sa[s]).wait()
        pltpu.make_async_copy(zb[s], acc_sh.at[idz[s]], sa[s]).wait()

    def _compute(p, s):
        for b in (0, 8):
            dv = ird[s][pl.ds(b, 16)]
            ids[s][pl.ds(b, 16)] = dv
            idz[s][pl.ds(b, 16)] = (dv >> 3) + _NPW

        def edge(e, c):
            live = jnp.where(p * _C + e < _EW, 1.0, 0.0)
            sv = zeros16
            for h in range(_H):
                sl = pl.ds(h * _D, _D)
                sc = (jnp.clip(kvb[s][e, sl] * qb[s][e, sl] * 0.25, -5.0, 5.0)
                      * eo[s][e, sl])
                eo[s][e, sl] = sc
                w = jnp.exp(jnp.clip(_allsum(sc), -5.0, 5.0)) * live
                qb[s][e, sl] = kvb[s][e, pl.ds(_HD + h * _D, _D)] * w
                sv = jnp.where(lane == h, w, sv)
            kvb[s][e, pl.ds(0, 16)] = sv
            return c
        lax.fori_loop(0, _C, edge, 0)

        for b in (0, 8):
            dvec = ids[s][pl.ds(b, 16)] & 7
            for j in range(16):
                e = b + j
                m = dvec[j]
                sv = kvb[s][e, pl.ds(0, 16)]
                for blk in range(8):
                    zb[s][e, pl.ds(blk * 16, 16)] = jnp.where(
                        m == blk, sv, zeros16)

    def _chunk_step(p, s):
        _wait_gathers(s)
        _compute(p, s)
        _issue_outputs(p, s)

        @pl.when(p > 0)
        def _():
            _wait_outputs(p - 1, 1 - s)

        @pl.when(p < _NCHUNK - 1)
        def _():
            _wait_idx(1 - s)
            _issue_gathers(p + 1, 1 - s)

        @pl.when(p < _NCHUNK - 2)
        def _():
            _issue_idx(p + 2, s)

    pb0 = pl.multiple_of(wid * _EWP, 8)
    pltpu.sync_copy(src_hbm.at[pl.ds(pb0, _C)], irs0)
    pltpu.sync_copy(dst_hbm.at[pl.ds(pb0, _C)], ird0)
    _issue_gathers(0, 0)
    _issue_idx(1, 1)

    def body(t, carry):
        _chunk_step(2 * t, 0)
        _chunk_step(2 * t + 1, 1)
        return carry
    lax.fori_loop(0, _NCHUNK // 2, body, 0)

    _wait_outputs(_NCHUNK - 1, 1)

    plsc.subcore_barrier()
    out0 = pl.multiple_of(cid * _NACC + row0, 8)

    def _out(k, c2):
        r0 = pl.multiple_of(row0 + k * _C, 8)
        o0 = pl.multiple_of(out0 + k * _C, 8)
        pltpu.sync_copy(acc_sh.at[pl.ds(r0, _C)], qb0)
        pltpu.sync_copy(qb0, acc_hbm.at[pl.ds(o0, _C)])
        return c2
    lax.fori_loop(0, _RPT // _C, _out, 0)


_edge_attention = functools.partial(
    pl.kernel,
    out_type=(
        jax.ShapeDtypeStruct((_E, _HD), jnp.float32),
        jax.ShapeDtypeStruct((_NC * _NACC, _HD), jnp.float32),
    ),
    mesh=plsc.VectorSubcoreMesh(core_axis_name="c", subcore_axis_name="s"),
    scratch_types=[
        pltpu.VMEM((_C,), jnp.int32),
        pltpu.VMEM((_C,), jnp.int32),
        pltpu.VMEM((_C,), jnp.int32),
        pltpu.VMEM((_C,), jnp.int32),
        pltpu.VMEM((_C,), jnp.int32),
        pltpu.VMEM((_C,), jnp.int32),
        pltpu.VMEM((_C,), jnp.int32),
        pltpu.VMEM((_C,), jnp.int32),
        pltpu.VMEM((_C, 2 * _HD), jnp.float32),
        pltpu.VMEM((_C, _HD), jnp.float32),
        pltpu.VMEM((_C, _HD), jnp.float32),
        pltpu.VMEM((_C, _HD), jnp.float32),
        pltpu.VMEM((_C, 2 * _HD), jnp.float32),
        pltpu.VMEM((_C, _HD), jnp.float32),
        pltpu.VMEM((_C, _HD), jnp.float32),
        pltpu.VMEM((_C, _HD), jnp.float32),
        pltpu.SemaphoreType.DMA,
        pltpu.SemaphoreType.DMA,
        pltpu.SemaphoreType.DMA,
        pltpu.SemaphoreType.DMA,
        pltpu.SemaphoreType.DMA,
        pltpu.SemaphoreType.DMA,
        pltpu.SemaphoreType.DMA,
        pltpu.SemaphoreType.DMA,
        pltpu.VMEM_SHARED((_NACC, _HD), jnp.float32),
    ],
)(_edge_kernel)


def _combine_body(wv_ref, z_ref, ex_ref, o_ref):
    wv = wv_ref[0] + wv_ref[1]
    zz = z_ref[0] + z_ref[1]
    den = jnp.dot(zz, ex_ref[...], preferred_element_type=jnp.float32) + 1e-6
    o_ref[...] = wv / den


def _combine(wv_part, z_part, expand):
    blk = 2000
    grid = _N // blk
    return pl.pallas_call(
        _combine_body,
        grid=(grid,),
        in_specs=[
            pl.BlockSpec((2, blk, _HD), lambda i: (0, i, 0)),
            pl.BlockSpec((2, blk, _H), lambda i: (0, i, 0)),
            pl.BlockSpec((_H, _HD), lambda i: (0, 0)),
        ],
        out_specs=pl.BlockSpec((blk, _HD), lambda i: (i, 0)),
        out_shape=jax.ShapeDtypeStruct((_N, _HD), jnp.float32),
    )(wv_part, z_part, expand)


def kernel(node_feats, edge_feats, edge_index, Wq, Wk, Wv, We):
    src_p = jnp.pad(edge_index[0].reshape(_NW, _EW),
                    ((0, 0), (0, _EWP - _EW))).reshape(-1)
    dst_p = jnp.pad(edge_index[1].reshape(_NW, _EW),
                    ((0, 0), (0, _EWP - _EW))).reshape(-1)
    wkv = jnp.concatenate([Wk, Wv], axis=1)

    kv, q = _node_proj(node_feats, wkv, Wq)
    pe = _edge_proj(edge_feats, We)

    e_out, acc = _edge_attention(kv, q, pe, src_p, dst_p)

    acc = acc.reshape(2, _NACC, _HD)
    wv_part = acc[:, :_N]
    z_part = (acc[:, _NPW:_NPW + _N // 8]
              .reshape(2, _N // 8, 8, 16)[:, :, :, :_H]
              .reshape(2, _N, _H))
    expand = (jnp.arange(_HD)[None, :] // _D == jnp.arange(_H)[:, None]
              ).astype(jnp.float32)
    h_out = _combine(wv_part, z_part, expand)

    return h_out.reshape(_N, _H, _D), e_out.reshape(_E, _H, _D)


# speedup vs baseline: 18.5550x; 1.4481x over previous
import functools

import jax
import jax.numpy as jnp
from jax import lax
from jax.experimental import pallas as pl
from jax.experimental.pallas import tpu as pltpu
from jax.experimental.pallas import tpu_sc as plsc

_N = 10000
_E = 320000
_F = 128
_H = 8
_D = 16
_HD = _H * _D

_NC = 2
_NS = 16
_NW = _NC * _NS
_EW = _E // _NW
_C = 24
_NCHUNK = 424
_TAIL = 416
_EWP = _NCHUNK * _C
_EPAD = 322000
_NPW = 10240
_NPZ = _NPW // 8
_NACC = _NPW + _NPZ
_RPT = _NACC // _NS


def _nproj_body(x_ref, wkv_ref, wq_ref, kv_ref, q_ref):
    x = x_ref[...]
    kv_ref[...] = jnp.dot(x, wkv_ref[...], preferred_element_type=jnp.float32)
    q_ref[...] = jnp.dot(x, wq_ref[...],
                         preferred_element_type=jnp.float32) * 0.25


def _node_proj(node_feats, wkv, wq):
    blk = 2000
    grid = _N // blk
    return pl.pallas_call(
        _nproj_body,
        grid=(grid,),
        in_specs=[
            pl.BlockSpec((blk, _F), lambda i: (i, 0)),
            pl.BlockSpec((_F, 2 * _HD), lambda i: (0, 0)),
            pl.BlockSpec((_F, _HD), lambda i: (0, 0)),
        ],
        out_specs=[
            pl.BlockSpec((blk, 2 * _HD), lambda i: (i, 0)),
            pl.BlockSpec((blk, _HD), lambda i: (i, 0)),
        ],
        out_shape=[
            jax.ShapeDtypeStruct((_N, 2 * _HD), jnp.float32),
            jax.ShapeDtypeStruct((_N, _HD), jnp.float32),
        ],
    )(node_feats, wkv, wq)


def _eproj_body(x_ref, w_ref, o_ref):
    o_ref[...] = jnp.dot(x_ref[...], w_ref[...], preferred_element_type=jnp.float32)


def _edge_proj(edge_feats, we):
    blk = 2000
    grid = _EPAD // blk
    return pl.pallas_call(
        _eproj_body,
        grid=(grid,),
        in_specs=[
            pl.BlockSpec((blk, _F), lambda i: (jnp.minimum(i, _E // blk - 1), 0)),
            pl.BlockSpec((_F, _HD), lambda i: (0, 0)),
        ],
        out_specs=pl.BlockSpec((blk, _HD), lambda i: (i, 0)),
        out_shape=jax.ShapeDtypeStruct((_EPAD, _HD), jnp.float32),
    )(edge_feats, we)


def _edge_kernel(kv_hbm, q_hbm, pe_hbm, src_hbm, dst_hbm,
                 eout_hbm, acc_hbm,
                 irs0, ird0, irs1, ird1,
                 ids0, idz0, ids1, idz1,
                 kvb0, qb0, eo0, zb0,
                 kvb1, qb1, eo1, zb1,
                 sg0, sg1, so0, so1, si0, si1, sa0, sa1,
                 acc_sh):
    cid = lax.axis_index("c")
    sid = lax.axis_index("s")
    wid = sid * _NC + cid
    row0 = pl.multiple_of(sid * _RPT, 8)

    irs = [irs0, irs1]
    ird = [ird0, ird1]
    ids = [ids0, ids1]
    idz = [idz0, idz1]
    kvb = [kvb0, kvb1]
    qb = [qb0, qb1]
    eo = [eo0, eo1]
    zb = [zb0, zb1]
    sg = [sg0, sg1]
    so = [so0, so1]
    si = [si0, si1]
    sa = [sa0, sa1]

    lane = lax.iota(jnp.int32, 16)
    zeros16 = jnp.zeros((16,), jnp.float32)
    p8, p4, p2, p1 = [(lane ^ k).reshape(16, 1) for k in (8, 4, 2, 1)]
    dnums = lax.GatherDimensionNumbers(
        offset_dims=(), collapsed_slice_dims=(0,), start_index_map=(0,))

    def _g(v, p):
        return lax.gather(v, p, dnums, (1,),
                          mode=lax.GatherScatterMode.PROMISE_IN_BOUNDS)

    _hlane = (0, 8, 4, 12, 2, 10, 6, 14)
    perm_sv = (((lane & 1) << 3) | ((lane & 2) << 1)
               | ((lane & 4) >> 1)).reshape(16, 1)
    bcast = [jnp.full((16, 1), _hlane[h], jnp.int32) for h in range(_H)]

    def _zrow(r, c2):
        for cc in range(8):
            qb0[r, pl.ds(cc * 16, 16)] = zeros16
        return c2
    lax.fori_loop(0, _C, _zrow, 0)

    def _init(k, c2):
        r0 = pl.multiple_of(row0 + k * _C, 8)
        pltpu.sync_copy(qb0, acc_sh.at[pl.ds(r0, _C)])
        return c2
    lax.fori_loop(0, _RPT // _C, _init, 0)
    plsc.subcore_barrier()

    def _issue_idx(p, slot):
        pb = pl.multiple_of(wid * _EWP + p * _C, 8)
        pltpu.async_copy(src_hbm.at[pl.ds(pb, _C)], irs[slot], si[slot])
        pltpu.async_copy(dst_hbm.at[pl.ds(pb, _C)], ird[slot], si[slot])

    def _wait_idx(slot):
        pb0 = pl.multiple_of(wid * _EWP, 8)
        pltpu.make_async_copy(src_hbm.at[pl.ds(pb0, _C)], irs[slot],
                              si[slot]).wait()
        pltpu.make_async_copy(dst_hbm.at[pl.ds(pb0, _C)], ird[slot],
                              si[slot]).wait()

    def _issue_gathers(p, s):
        rb = pl.multiple_of(wid * _EW + jnp.minimum(p, _TAIL) * _C, 8)
        pltpu.async_copy(kv_hbm.at[irs[s]], kvb[s], sg[s])
        pltpu.async_copy(q_hbm.at[ird[s]], qb[s], sg[s])
        pltpu.async_copy(pe_hbm.at[pl.ds(rb, _C)], eo[s], sg[s])

    def _wait_gathers(s):
        rb0 = pl.multiple_of(wid * _EW, 8)
        pltpu.make_async_copy(kv_hbm.at[irs[s]], kvb[s], sg[s]).wait()
        pltpu.make_async_copy(q_hbm.at[ird[s]], qb[s], sg[s]).wait()
        pltpu.make_async_copy(pe_hbm.at[pl.ds(rb0, _C)], eo[s], sg[s]).wait()

    def _issue_outputs(p, s):
        rb = pl.multiple_of(wid * _EW + jnp.minimum(p, _TAIL) * _C, 8)

        @pl.when(p < _TAIL)
        def _():
            pltpu.async_copy(eo[s], eout_hbm.at[pl.ds(rb, _C)], so[s])

        @pl.when(p == _TAIL)
        def _():
            pltpu.async_copy(eo[s].at[pl.ds(0, 16)],
                             eout_hbm.at[pl.ds(rb, 16)], so[s])

        pltpu.async_copy(qb[s], acc_sh.at[ids[s]], sa[s], add=True)
        pltpu.async_copy(zb[s], acc_sh.at[idz[s]], sa[s], add=True)

    def _wait_outputs(p, s):
        rb0 = pl.multiple_of(wid * _EW, 8)

        @pl.when(p < _TAIL)
        def _():
            pltpu.make_async_copy(eo[s], eout_hbm.at[pl.ds(rb0, _C)],
                                  so[s]).wait()

        @pl.when(p == _TAIL)
        def _():
            pltpu.make_async_copy(eo[s].at[pl.ds(0, 16)],
                                  eout_hbm.at[pl.ds(rb0, 16)], so[s]).wait()

        pltpu.make_async_copy(qb[s], acc_sh.at[ids[s]], sa[s]).wait()
        pltpu.make_async_copy(zb[s], acc_sh.at[idz[s]], sa[s]).wait()

    def _compute(p, s):
        for b in (0, 8):
            dv = ird[s][pl.ds(b, 16)]
            ids[s][pl.ds(b, 16)] = dv
            idz[s][pl.ds(b, 16)] = (dv >> 3) + _NPW

        def edge(e, c):
            live = jnp.where(p * _C + e < _EW, 1.0, 0.0)
            sc = []
            for h in range(_H):
                sl = pl.ds(h * _D, _D)
                s2 = (jnp.clip(kvb[s][e, sl] * qb[s][e, sl], -5.0, 5.0)
                      * eo[s][e, sl])
                eo[s][e, sl] = s2
                sc.append(s2)
            t = []
            for k in range(4):
                a, b = sc[2 * k], sc[2 * k + 1]
                t.append(jnp.where(lane < 8, a + _g(a, p8), b + _g(b, p8)))
            u = []
            for k in range(2):
                a = t[2 * k] + _g(t[2 * k], p4)
                b = t[2 * k + 1] + _g(t[2 * k + 1], p4)
                u.append(jnp.where((lane & 4) == 0, a, _g(b, p4)))
            a = u[0] + _g(u[0], p2)
            b = u[1] + _g(u[1], p2)
            v2 = jnp.where((lane & 2) == 0, a, _g(b, p2))
            f = v2 + _g(v2, p1)
            wv = jnp.exp(jnp.clip(f, -5.0, 5.0)) * live
            kvb[s][e, pl.ds(0, 16)] = _g(wv, perm_sv)
            for h in range(_H):
                qb[s][e, pl.ds(h * _D, _D)] = (
                    kvb[s][e, pl.ds(_HD + h * _D, _D)] * _g(wv, bcast[h]))
            return c
        lax.fori_loop(0, _C, edge, 0)

        for b in (0, 8):
            dvec = ids[s][pl.ds(b, 16)] & 7
            for j in range(16):
                e = b + j
                m = dvec[j]
                sv = kvb[s][e, pl.ds(0, 16)]
                for blk in range(8):
                    zb[s][e, pl.ds(blk * 16, 16)] = jnp.where(
                        m == blk, sv, zeros16)

    def _chunk_step(p, s):
        _wait_gathers(s)
        _compute(p, s)
        _issue_outputs(p, s)

        @pl.when(p > 0)
        def _():
            _wait_outputs(p - 1, 1 - s)

        @pl.when(p < _NCHUNK - 1)
        def _():
            _wait_idx(1 - s)
            _issue_gathers(p + 1, 1 - s)

        @pl.when(p < _NCHUNK - 2)
        def _():
            _issue_idx(p + 2, s)

    pb0 = pl.multiple_of(wid * _EWP, 8)
    pltpu.sync_copy(src_hbm.at[pl.ds(pb0, _C)], irs0)
    pltpu.sync_copy(dst_hbm.at[pl.ds(pb0, _C)], ird0)
    _issue_gathers(0, 0)
    _issue_idx(1, 1)

    def body(t, carry):
        _chunk_step(2 * t, 0)
        _chunk_step(2 * t + 1, 1)
        return carry
    lax.fori_loop(0, _NCHUNK // 2, body, 0)

    _wait_outputs(_NCHUNK - 1, 1)

    plsc.subcore_barrier()
    out0 = pl.multiple_of(cid * _NACC + row0, 8)

    def _out(k, c2):
        r0 = pl.multiple_of(row0 + k * _C, 8)
        o0 = pl.multiple_of(out0 + k * _C, 8)
        pltpu.sync_copy(acc_sh.at[pl.ds(r0, _C)], qb0)
        pltpu.sync_copy(qb0, acc_hbm.at[pl.ds(o0, _C)])
        return c2
    lax.fori_loop(0, _RPT // _C, _out, 0)


_edge_attention = functools.partial(
    pl.kernel,
    out_type=(
        jax.ShapeDtypeStruct((_E, _HD), jnp.float32),
        jax.ShapeDtypeStruct((_NC * _NACC, _HD), jnp.float32),
    ),
    mesh=plsc.VectorSubcoreMesh(core_axis_name="c", subcore_axis_name="s"),
    scratch_types=[
        pltpu.VMEM((_C,), jnp.int32),
        pltpu.VMEM((_C,), jnp.int32),
        pltpu.VMEM((_C,), jnp.int32),
        pltpu.VMEM((_C,), jnp.int32),
        pltpu.VMEM((_C,), jnp.int32),
        pltpu.VMEM((_C,), jnp.int32),
        pltpu.VMEM((_C,), jnp.int32),
        pltpu.VMEM((_C,), jnp.int32),
        pltpu.VMEM((_C, 2 * _HD), jnp.float32),
        pltpu.VMEM((_C, _HD), jnp.float32),
        pltpu.VMEM((_C, _HD), jnp.float32),
        pltpu.VMEM((_C, _HD), jnp.float32),
        pltpu.VMEM((_C, 2 * _HD), jnp.float32),
        pltpu.VMEM((_C, _HD), jnp.float32),
        pltpu.VMEM((_C, _HD), jnp.float32),
        pltpu.VMEM((_C, _HD), jnp.float32),
        pltpu.SemaphoreType.DMA,
        pltpu.SemaphoreType.DMA,
        pltpu.SemaphoreType.DMA,
        pltpu.SemaphoreType.DMA,
        pltpu.SemaphoreType.DMA,
        pltpu.SemaphoreType.DMA,
        pltpu.SemaphoreType.DMA,
        pltpu.SemaphoreType.DMA,
        pltpu.VMEM_SHARED((_NACC, _HD), jnp.float32),
    ],
)(_edge_kernel)


def _combine_body(wv_ref, z_ref, ex_ref, o_ref):
    wv = wv_ref[0] + wv_ref[1]
    zz = z_ref[0] + z_ref[1]
    den = jnp.dot(zz, ex_ref[...], preferred_element_type=jnp.float32) + 1e-6
    o_ref[...] = wv / den


def _combine(wv_part, z_part, expand):
    blk = 2000
    grid = _N // blk
    return pl.pallas_call(
        _combine_body,
        grid=(grid,),
        in_specs=[
            pl.BlockSpec((2, blk, _HD), lambda i: (0, i, 0)),
            pl.BlockSpec((2, blk, _H), lambda i: (0, i, 0)),
            pl.BlockSpec((_H, _HD), lambda i: (0, 0)),
        ],
        out_specs=pl.BlockSpec((blk, _HD), lambda i: (i, 0)),
        out_shape=jax.ShapeDtypeStruct((_N, _HD), jnp.float32),
    )(wv_part, z_part, expand)


def kernel(node_feats, edge_feats, edge_index, Wq, Wk, Wv, We):
    src_p = jnp.pad(edge_index[0].reshape(_NW, _EW),
                    ((0, 0), (0, _EWP - _EW))).reshape(-1)
    dst_p = jnp.pad(edge_index[1].reshape(_NW, _EW),
                    ((0, 0), (0, _EWP - _EW))).reshape(-1)
    wkv = jnp.concatenate([Wk, Wv], axis=1)

    kv, q = _node_proj(node_feats, wkv, Wq)
    pe = _edge_proj(edge_feats, We)

    e_out, acc = _edge_attention(kv, q, pe, src_p, dst_p)

    acc = acc.reshape(2, _NACC, _HD)
    wv_part = acc[:, :_N]
    z_part = (acc[:, _NPW:_NPW + _N // 8]
              .reshape(2, _N // 8, 8, 16)[:, :, :, :_H]
              .reshape(2, _N, _H))
    expand = (jnp.arange(_HD)[None, :] // _D == jnp.arange(_H)[:, None]
              ).astype(jnp.float32)
    h_out = _combine(wv_part, z_part, expand)

    return h_out.reshape(_N, _H, _D), e_out.reshape(_E, _H, _D)


# speedup vs baseline: 18.6653x; 1.0059x over previous
import functools

import jax
import jax.numpy as jnp
from jax import lax
from jax.experimental import pallas as pl
from jax.experimental.pallas import tpu as pltpu
from jax.experimental.pallas import tpu_sc as plsc

_N = 10000
_E = 320000
_F = 128
_H = 8
_D = 16
_HD = _H * _D

_NC = 2
_NS = 16
_NW = _NC * _NS
_EW = _E // _NW
_C = 24
_NCHUNK = 424
_TAIL = 416
_EWP = _NCHUNK * _C
_EPAD = 322000
_NPW = 10240
_NPZ = _NPW // 8
_NACC = _NPW + _NPZ
_RPT = _NACC // _NS


def _nproj_body(x_ref, wkv_ref, wq_ref, kv_ref, q_ref):
    x = x_ref[...]
    kv_ref[...] = jnp.dot(x, wkv_ref[...], preferred_element_type=jnp.float32)
    q_ref[...] = jnp.dot(x, wq_ref[...],
                         preferred_element_type=jnp.float32) * 0.25


def _node_proj(node_feats, wkv, wq):
    blk = 2000
    grid = _N // blk
    return pl.pallas_call(
        _nproj_body,
        grid=(grid,),
        in_specs=[
            pl.BlockSpec((blk, _F), lambda i: (i, 0)),
            pl.BlockSpec((_F, 2 * _HD), lambda i: (0, 0)),
            pl.BlockSpec((_F, _HD), lambda i: (0, 0)),
        ],
        out_specs=[
            pl.BlockSpec((blk, 2 * _HD), lambda i: (i, 0)),
            pl.BlockSpec((blk, _HD), lambda i: (i, 0)),
        ],
        out_shape=[
            jax.ShapeDtypeStruct((_N, 2 * _HD), jnp.float32),
            jax.ShapeDtypeStruct((_N, _HD), jnp.float32),
        ],
    )(node_feats, wkv, wq)


def _eproj_body(x_ref, w_ref, o_ref):
    o_ref[...] = jnp.dot(x_ref[...], w_ref[...], preferred_element_type=jnp.float32)


def _edge_proj(edge_feats, we):
    blk = 2000
    grid = _EPAD // blk
    return pl.pallas_call(
        _eproj_body,
        grid=(grid,),
        in_specs=[
            pl.BlockSpec((blk, _F), lambda i: (jnp.minimum(i, _E // blk - 1), 0)),
            pl.BlockSpec((_F, _HD), lambda i: (0, 0)),
        ],
        out_specs=pl.BlockSpec((blk, _HD), lambda i: (i, 0)),
        out_shape=jax.ShapeDtypeStruct((_EPAD, _HD), jnp.float32),
    )(edge_feats, we)


def _edge_kernel(kv_hbm, q_hbm, pe_hbm, src_hbm, dst_hbm,
                 eout_hbm, acc_hbm,
                 irs0, ird0, irs1, ird1,
                 ids0, idz0, ids1, idz1,
                 kvb0, qb0, eo0, zb0,
                 kvb1, qb1, eo1, zb1,
                 sg0, sg1, so0, so1, si0, si1, sa0, sa1,
                 acc_sh):
    cid = lax.axis_index("c")
    sid = lax.axis_index("s")
    wid = sid * _NC + cid
    row0 = pl.multiple_of(sid * _RPT, 8)

    irs = [irs0, irs1]
    ird = [ird0, ird1]
    ids = [ids0, ids1]
    idz = [idz0, idz1]
    kvb = [kvb0, kvb1]
    qb = [qb0, qb1]
    eo = [eo0, eo1]
    zb = [zb0, zb1]
    sg = [sg0, sg1]
    so = [so0, so1]
    si = [si0, si1]
    sa = [sa0, sa1]

    lane = lax.iota(jnp.int32, 16)
    zeros16 = jnp.zeros((16,), jnp.float32)
    p8, p4, p2, p1 = [(lane ^ k).reshape(16, 1) for k in (8, 4, 2, 1)]
    dnums = lax.GatherDimensionNumbers(
        offset_dims=(), collapsed_slice_dims=(0,), start_index_map=(0,))

    def _g(v, p):
        return lax.gather(v, p, dnums, (1,),
                          mode=lax.GatherScatterMode.PROMISE_IN_BOUNDS)

    _hlane = (0, 8, 4, 12, 2, 10, 6, 14)
    perm_sv = (((lane & 1) << 3) | ((lane & 2) << 1)
               | ((lane & 4) >> 1)).reshape(16, 1)
    bcast = [jnp.full((16, 1), _hlane[h], jnp.int32) for h in range(_H)]

    def _zrow(r, c2):
        for cc in range(8):
            qb0[r, pl.ds(cc * 16, 16)] = zeros16
        return c2
    lax.fori_loop(0, _C, _zrow, 0)

    def _init(k, c2):
        r0 = pl.multiple_of(row0 + k * _C, 8)
        pltpu.sync_copy(qb0, acc_sh.at[pl.ds(r0, _C)])
        return c2
    lax.fori_loop(0, _RPT // _C, _init, 0)
    plsc.subcore_barrier()

    def _issue_idx(p, slot):
        pb = pl.multiple_of(wid * _EWP + p * _C, 8)
        pltpu.async_copy(src_hbm.at[pl.ds(pb, _C)], irs[slot], si[slot])
        pltpu.async_copy(dst_hbm.at[pl.ds(pb, _C)], ird[slot], si[slot])

    def _wait_idx(slot):
        pb0 = pl.multiple_of(wid * _EWP, 8)
        pltpu.make_async_copy(src_hbm.at[pl.ds(pb0, _C)], irs[slot],
                              si[slot]).wait()
        pltpu.make_async_copy(dst_hbm.at[pl.ds(pb0, _C)], ird[slot],
                              si[slot]).wait()

    def _issue_gathers(p, s):
        rb = pl.multiple_of(wid * _EW + jnp.minimum(p, _TAIL) * _C, 8)
        pltpu.async_copy(kv_hbm.at[irs[s]], kvb[s], sg[s])
        pltpu.async_copy(q_hbm.at[ird[s]], qb[s], sg[s])
        pltpu.async_copy(pe_hbm.at[pl.ds(rb, _C)], eo[s], sg[s])

    def _wait_gathers(s):
        rb0 = pl.multiple_of(wid * _EW, 8)
        pltpu.make_async_copy(kv_hbm.at[irs[s]], kvb[s], sg[s]).wait()
        pltpu.make_async_copy(q_hbm.at[ird[s]], qb[s], sg[s]).wait()
        pltpu.make_async_copy(pe_hbm.at[pl.ds(rb0, _C)], eo[s], sg[s]).wait()

    def _issue_outputs(p, s):
        rb = pl.multiple_of(wid * _EW + jnp.minimum(p, _TAIL) * _C, 8)

        @pl.when(p < _TAIL)
        def _():
            pltpu.async_copy(eo[s], eout_hbm.at[pl.ds(rb, _C)], so[s])

        @pl.when(p == _TAIL)
        def _():
            pltpu.async_copy(eo[s].at[pl.ds(0, 16)],
                             eout_hbm.at[pl.ds(rb, 16)], so[s])

        pltpu.async_copy(qb[s], acc_sh.at[ids[s]], sa[s], add=True)
        pltpu.async_copy(zb[s], acc_sh.at[idz[s]], sa[s], add=True)

    def _wait_outputs(p, s):
        rb0 = pl.multiple_of(wid * _EW, 8)

        @pl.when(p < _TAIL)
        def _():
            pltpu.make_async_copy(eo[s], eout_hbm.at[pl.ds(rb0, _C)],
                                  so[s]).wait()

        @pl.when(p == _TAIL)
        def _():
            pltpu.make_async_copy(eo[s].at[pl.ds(0, 16)],
                                  eout_hbm.at[pl.ds(rb0, 16)], so[s]).wait()

        pltpu.make_async_copy(qb[s], acc_sh.at[ids[s]], sa[s]).wait()
        pltpu.make_async_copy(zb[s], acc_sh.at[idz[s]], sa[s]).wait()

    def _compute(p, s):
        for b in (0, 8):
            dv = ird[s][pl.ds(b, 16)]
            ids[s][pl.ds(b, 16)] = dv
            idz[s][pl.ds(b, 16)] = (dv >> 3) + _NPW

        def edge(e, c):
            live = jnp.where(p * _C + e < _EW, 1.0, 0.0)
            sc = []
            for h in range(_H):
                sl = pl.ds(h * _D, _D)
                s2 = (jnp.clip(kvb[s][e, sl] * qb[s][e, sl], -5.0, 5.0)
                      * eo[s][e, sl])
                eo[s][e, sl] = s2
                sc.append(s2)
            t = []
            for k in range(4):
                a, b = sc[2 * k], sc[2 * k + 1]
                t.append(jnp.where(lane < 8, a + _g(a, p8), b + _g(b, p8)))
            u = []
            for k in range(2):
                a = t[2 * k] + _g(t[2 * k], p4)
                b = t[2 * k + 1] + _g(t[2 * k + 1], p4)
                u.append(jnp.where((lane & 4) == 0, a, _g(b, p4)))
            a = u[0] + _g(u[0], p2)
            b = u[1] + _g(u[1], p2)
            v2 = jnp.where((lane & 2) == 0, a, _g(b, p2))
            f = v2 + _g(v2, p1)
            wv = jnp.exp(jnp.clip(f, -5.0, 5.0)) * live
            kvb[s][e, pl.ds(0, 16)] = _g(wv, perm_sv)
            for h in range(_H):
                qb[s][e, pl.ds(h * _D, _D)] = (
                    kvb[s][e, pl.ds(_HD + h * _D, _D)] * _g(wv, bcast[h]))
            return c
        lax.fori_loop(0, _C, edge, 0)

        for b in (0, 8):
            dvec = ids[s][pl.ds(b, 16)] & 7
            for j in (range(16) if b == 0 else range(8, 16)):
                e = b + j
                m = dvec[j]
                sv = kvb[s][e, pl.ds(0, 16)]
                for blk in range(8):
                    zb[s][e, pl.ds(blk * 16, 16)] = jnp.where(
                        m == blk, sv, zeros16)

    def _chunk_step(p, s):
        _wait_gathers(s)
        _compute(p, s)
        _issue_outputs(p, s)

        @pl.when(p > 0)
        def _():
            _wait_outputs(p - 1, 1 - s)

        @pl.when(p < _NCHUNK - 1)
        def _():
            _wait_idx(1 - s)
            _issue_gathers(p + 1, 1 - s)

        @pl.when(p < _NCHUNK - 2)
        def _():
            _issue_idx(p + 2, s)

    pb0 = pl.multiple_of(wid * _EWP, 8)
    pltpu.sync_copy(src_hbm.at[pl.ds(pb0, _C)], irs0)
    pltpu.sync_copy(dst_hbm.at[pl.ds(pb0, _C)], ird0)
    _issue_gathers(0, 0)
    _issue_idx(1, 1)

    def body(t, carry):
        _chunk_step(2 * t, 0)
        _chunk_step(2 * t + 1, 1)
        return carry
    lax.fori_loop(0, _NCHUNK // 2, body, 0)

    _wait_outputs(_NCHUNK - 1, 1)

    plsc.subcore_barrier()
    out0 = pl.multiple_of(cid * _NACC + row0, 8)

    def _out(k, c2):
        r0 = pl.multiple_of(row0 + k * _C, 8)
        o0 = pl.multiple_of(out0 + k * _C, 8)
        pltpu.sync_copy(acc_sh.at[pl.ds(r0, _C)], qb0)
        pltpu.sync_copy(qb0, acc_hbm.at[pl.ds(o0, _C)])
        return c2
    lax.fori_loop(0, _RPT // _C, _out, 0)


_edge_attention = functools.partial(
    pl.kernel,
    out_type=(
        jax.ShapeDtypeStruct((_E, _HD), jnp.float32),
        jax.ShapeDtypeStruct((_NC * _NACC, _HD), jnp.float32),
    ),
    mesh=plsc.VectorSubcoreMesh(core_axis_name="c", subcore_axis_name="s"),
    scratch_types=[
        pltpu.VMEM((_C,), jnp.int32),
        pltpu.VMEM((_C,), jnp.int32),
        pltpu.VMEM((_C,), jnp.int32),
        pltpu.VMEM((_C,), jnp.int32),
        pltpu.VMEM((_C,), jnp.int32),
        pltpu.VMEM((_C,), jnp.int32),
        pltpu.VMEM((_C,), jnp.int32),
        pltpu.VMEM((_C,), jnp.int32),
        pltpu.VMEM((_C, 2 * _HD), jnp.float32),
        pltpu.VMEM((_C, _HD), jnp.float32),
        pltpu.VMEM((_C, _HD), jnp.float32),
        pltpu.VMEM((_C, _HD), jnp.float32),
        pltpu.VMEM((_C, 2 * _HD), jnp.float32),
        pltpu.VMEM((_C, _HD), jnp.float32),
        pltpu.VMEM((_C, _HD), jnp.float32),
        pltpu.VMEM((_C, _HD), jnp.float32),
        pltpu.SemaphoreType.DMA,
        pltpu.SemaphoreType.DMA,
        pltpu.SemaphoreType.DMA,
        pltpu.SemaphoreType.DMA,
        pltpu.SemaphoreType.DMA,
        pltpu.SemaphoreType.DMA,
        pltpu.SemaphoreType.DMA,
        pltpu.SemaphoreType.DMA,
        pltpu.VMEM_SHARED((_NACC, _HD), jnp.float32),
    ],
)(_edge_kernel)


def _combine_body(wv_ref, z_ref, ex_ref, o_ref):
    wv = wv_ref[0] + wv_ref[1]
    zz = z_ref[0] + z_ref[1]
    den = jnp.dot(zz, ex_ref[...], preferred_element_type=jnp.float32) + 1e-6
    o_ref[...] = wv / den


def _combine(wv_part, z_part, expand):
    blk = 2000
    grid = _N // blk
    return pl.pallas_call(
        _combine_body,
        grid=(grid,),
        in_specs=[
            pl.BlockSpec((2, blk, _HD), lambda i: (0, i, 0)),
            pl.BlockSpec((2, blk, _H), lambda i: (0, i, 0)),
            pl.BlockSpec((_H, _HD), lambda i: (0, 0)),
        ],
        out_specs=pl.BlockSpec((blk, _HD), lambda i: (i, 0)),
        out_shape=jax.ShapeDtypeStruct((_N, _HD), jnp.float32),
    )(wv_part, z_part, expand)


def kernel(node_feats, edge_feats, edge_index, Wq, Wk, Wv, We):
    src_p = jnp.pad(edge_index[0].reshape(_NW, _EW),
                    ((0, 0), (0, _EWP - _EW))).reshape(-1)
    dst_p = jnp.pad(edge_index[1].reshape(_NW, _EW),
                    ((0, 0), (0, _EWP - _EW))).reshape(-1)
    wkv = jnp.concatenate([Wk, Wv], axis=1)

    kv, q = _node_proj(node_feats, wkv, Wq)
    pe = _edge_proj(edge_feats, We)

    e_out, acc = _edge_attention(kv, q, pe, src_p, dst_p)

    acc = acc.reshape(2, _NACC, _HD)
    wv_part = acc[:, :_N]
    z_part = (acc[:, _NPW:_NPW + _N // 8]
              .reshape(2, _N // 8, 8, 16)[:, :, :, :_H]
              .reshape(2, _N, _H))
    expand = (jnp.arange(_HD)[None, :] // _D == jnp.arange(_H)[:, None]
              ).astype(jnp.float32)
    h_out = _combine(wv_part, z_part, expand)

    return h_out.reshape(_N, _H, _D), e_out.reshape(_E, _H, _D)
